# Initial kernel scaffold; baseline (speedup 1.0000x reference)
import jax
import jax.numpy as jnp
from jax.experimental import pallas as pl


def kernel(self_feat, nbr_feat, relation_src_indices, W):
    raise NotImplementedError("write your pallas kernel here")


# speedup vs baseline: 5.8020x; 5.8020x over previous
import functools

import jax
import jax.numpy as jnp
from jax import lax
from jax.experimental import pallas as pl
from jax.experimental.pallas import tpu as pltpu
from jax.experimental.pallas import tpu_sc as plsc

N_NODES = 10000
N_EDGES = 320000
D = 128

NC = 2
NS = 16
LANES = 16

EDGES_PER_CORE = N_EDGES // NC
EDGES_PER_TILE = EDGES_PER_CORE // NS
SCAT = 80
CHUNK = 80
KSUB = CHUNK // SCAT
NCHUNK = EDGES_PER_TILE // CHUNK
IDX_ROWS = EDGES_PER_TILE // SCAT
N_PAD = 10240
ROWS_PER_TILE = N_PAD // NS
CNT_PAD = 10240
CNT_PER_TILE = CNT_PAD // NS


def _sc_segment_sum(nbr_feat, idx2d):
    mesh = plsc.VectorSubcoreMesh(core_axis_name="c", subcore_axis_name="s")

    @functools.partial(
        pl.kernel,
        out_type=(
            jax.ShapeDtypeStruct((NC, N_PAD, D), jnp.float32),
            jax.ShapeDtypeStruct((NC, CNT_PAD), jnp.float32),
        ),
        mesh=mesh,
        scratch_types=[
            pltpu.VMEM((CHUNK, D), jnp.float32),
            pltpu.VMEM((IDX_ROWS, SCAT), jnp.int32),
            pltpu.VMEM((SCAT,), jnp.float32),
            pltpu.VMEM((CNT_PER_TILE,), jnp.float32),
            pltpu.VMEM_SHARED((N_PAD, D), jnp.float32),
            pltpu.VMEM_SHARED((CNT_PAD,), jnp.float32),
        ],
    )
    def sc_kernel(nbr_hbm, idx_hbm, sums_out, cnt_out,
                  rows_v, idx_v, ones_v, zcnt_v, acc_sh, cnt_sh):
        cid = lax.axis_index("c")
        sid = lax.axis_index("s")

        zeros16 = jnp.zeros((LANES,), jnp.float32)
        ones16 = jnp.ones((LANES,), jnp.float32)

        def zero_row(r, _):
            for c in range(D // LANES):
                rows_v[r, pl.ds(c * LANES, LANES)] = zeros16
            return 0

        lax.fori_loop(0, CHUNK, zero_row, 0)

        def zero_cnt(i, _):
            zcnt_v[pl.ds(i * LANES, LANES)] = zeros16
            return 0

        lax.fori_loop(0, CNT_PER_TILE // LANES, zero_cnt, 0)

        for i in range(SCAT // LANES):
            ones_v[pl.ds(i * LANES, LANES)] = ones16

        row0 = sid * ROWS_PER_TILE

        def zero_acc(t, _):
            pltpu.sync_copy(rows_v, acc_sh.at[pl.ds(row0 + t * CHUNK, CHUNK)])
            return 0

        lax.fori_loop(0, ROWS_PER_TILE // CHUNK, zero_acc, 0)
        pltpu.sync_copy(zcnt_v, cnt_sh.at[pl.ds(sid * CNT_PER_TILE, CNT_PER_TILE)])

        plsc.subcore_barrier()

        wid = cid * NS + sid
        edge_base = wid * EDGES_PER_TILE
        pltpu.sync_copy(idx_hbm.at[wid], idx_v)

        def body(j, _):
            pltpu.sync_copy(nbr_hbm.at[pl.ds(edge_base + j * CHUNK, CHUNK)],
                            rows_v)
            for k in range(KSUB):
                pltpu.sync_copy(rows_v.at[pl.ds(k * SCAT, SCAT)],
                                acc_sh.at[idx_v.at[j * KSUB + k]], add=True)
                pltpu.sync_copy(ones_v, cnt_sh.at[idx_v.at[j * KSUB + k]],
                                add=True)
            return 0

        lax.fori_loop(0, NCHUNK, body, 0)

        plsc.subcore_barrier()

        pltpu.sync_copy(acc_sh.at[pl.ds(row0, ROWS_PER_TILE)],
                        sums_out.at[cid, pl.ds(row0, ROWS_PER_TILE)])
        pltpu.sync_copy(cnt_sh.at[pl.ds(sid * CNT_PER_TILE, CNT_PER_TILE)],
                        cnt_out.at[cid, pl.ds(sid * CNT_PER_TILE, CNT_PER_TILE)])

    return sc_kernel(nbr_feat, idx2d)


def _tc_combine(self_feat, sums_p, cnt_p, W1, W2):
    BLK = 1024
    grid = (N_NODES + BLK - 1) // BLK

    def tc_kernel(self_ref, sums_ref, cnt_ref, w1_ref, w2_ref, out_ref):
        pid = pl.program_id(0)
        s = sums_ref[0] + sums_ref[1]
        c = (cnt_ref[0, pl.ds(pid * BLK, BLK)]
             + cnt_ref[1, pl.ds(pid * BLK, BLK)])
        denom = jnp.maximum(c, 1.0)
        agg = s * (1.0 / denom)[:, None]
        dn = (((1,), (1,)), ((), ()))
        out_ref[...] = (
            lax.dot_general(self_ref[...], w1_ref[...], dn,
                            preferred_element_type=jnp.float32)
            + lax.dot_general(agg, w2_ref[...], dn,
                              preferred_element_type=jnp.float32)
        )

    return pl.pallas_call(
        tc_kernel,
        out_shape=jax.ShapeDtypeStruct((N_NODES, D), jnp.float32),
        grid=(grid,),
        in_specs=[
            pl.BlockSpec((BLK, D), lambda i: (i, 0)),
            pl.BlockSpec((NC, BLK, D), lambda i: (0, i, 0)),
            pl.BlockSpec((NC, CNT_PAD), lambda i: (0, 0)),
            pl.BlockSpec((D, D), lambda i: (0, 0)),
            pl.BlockSpec((D, D), lambda i: (0, 0)),
        ],
        out_specs=pl.BlockSpec((BLK, D), lambda i: (i, 0)),
    )(self_feat, sums_p, cnt_p, W1, W2)


@jax.jit
def kernel(self_feat, nbr_feat, relation_src_indices, W):
    idx3d = relation_src_indices.astype(jnp.int32).reshape(
        NC * NS, IDX_ROWS, SCAT)
    sums_p, cnt_p = _sc_segment_sum(nbr_feat, idx3d)
    W1 = W[:, :D]
    W2 = W[:, D:]
    return _tc_combine(self_feat, sums_p, cnt_p, W1, W2)


# speedup vs baseline: 7.7144x; 1.3296x over previous
import functools

import jax
import jax.numpy as jnp
from jax import lax
from jax.experimental import pallas as pl
from jax.experimental.pallas import tpu as pltpu
from jax.experimental.pallas import tpu_sc as plsc

N_NODES = 10000
N_EDGES = 320000
D = 128

NC = 2
NS = 16
LANES = 16

EDGES_PER_CORE = N_EDGES // NC
EDGES_PER_TILE = EDGES_PER_CORE // NS
SCAT = 80
CHUNK = 80
KSUB = CHUNK // SCAT
NCHUNK = EDGES_PER_TILE // CHUNK
IDX_ROWS = EDGES_PER_TILE // SCAT
N_PAD = 10240
ROWS_PER_TILE = N_PAD // NS
CNT_PAD = 10240
CNT_PER_TILE = CNT_PAD // NS


def _sc_segment_sum(nbr_feat, idx2d):
    mesh = plsc.VectorSubcoreMesh(core_axis_name="c", subcore_axis_name="s")

    @functools.partial(
        pl.kernel,
        out_type=(
            jax.ShapeDtypeStruct((NC, N_PAD, D), jnp.float32),
            jax.ShapeDtypeStruct((NC, CNT_PAD), jnp.float32),
        ),
        mesh=mesh,
        scratch_types=[
            pltpu.VMEM((2, SCAT, D), jnp.float32),
            pltpu.VMEM((IDX_ROWS, SCAT), jnp.int32),
            pltpu.VMEM((SCAT,), jnp.float32),
            pltpu.VMEM((CNT_PER_TILE,), jnp.float32),
            pltpu.VMEM_SHARED((N_PAD, D), jnp.float32),
            pltpu.VMEM_SHARED((CNT_PAD,), jnp.float32),
            pltpu.SemaphoreType.DMA,
            pltpu.SemaphoreType.DMA,
            pltpu.SemaphoreType.DMA,
            pltpu.SemaphoreType.DMA,
        ],
    )
    def sc_kernel(nbr_hbm, idx_hbm, sums_out, cnt_out,
                  rows_v, idx_v, ones_v, zcnt_v, acc_sh, cnt_sh,
                  sem_s0, sem_s1, sem_c0, sem_c1):
        sem_s = [sem_s0, sem_s1]
        sem_c = [sem_c0, sem_c1]
        cid = lax.axis_index("c")
        sid = lax.axis_index("s")

        zeros16 = jnp.zeros((LANES,), jnp.float32)
        ones16 = jnp.ones((LANES,), jnp.float32)

        def zero_row(r, _):
            for c in range(D // LANES):
                rows_v[0, r, pl.ds(c * LANES, LANES)] = zeros16
            return 0

        lax.fori_loop(0, SCAT, zero_row, 0)

        def zero_cnt(i, _):
            zcnt_v[pl.ds(i * LANES, LANES)] = zeros16
            return 0

        lax.fori_loop(0, CNT_PER_TILE // LANES, zero_cnt, 0)

        for i in range(SCAT // LANES):
            ones_v[pl.ds(i * LANES, LANES)] = ones16

        row0 = sid * ROWS_PER_TILE

        def zero_acc(t, _):
            pltpu.sync_copy(rows_v.at[0],
                            acc_sh.at[pl.ds(row0 + t * SCAT, SCAT)])
            return 0

        lax.fori_loop(0, ROWS_PER_TILE // SCAT, zero_acc, 0)
        pltpu.sync_copy(zcnt_v, cnt_sh.at[pl.ds(sid * CNT_PER_TILE, CNT_PER_TILE)])

        plsc.subcore_barrier()

        wid = cid * NS + sid
        edge_base = wid * EDGES_PER_TILE
        pltpu.sync_copy(idx_hbm.at[wid], idx_v)

        NCH = EDGES_PER_TILE // SCAT

        def drain_slot(t):
            pltpu.make_async_copy(rows_v.at[t], acc_sh.at[pl.ds(0, SCAT)],
                                  sem_s[t]).wait()
            pltpu.make_async_copy(ones_v, cnt_sh.at[pl.ds(0, SCAT)],
                                  sem_c[t]).wait()

        def fire_chunk(j, t):
            pltpu.sync_copy(nbr_hbm.at[pl.ds(edge_base + j * SCAT, SCAT)],
                            rows_v.at[t])
            pltpu.async_copy(rows_v.at[t], acc_sh.at[idx_v.at[j]],
                             sem_s[t], add=True)
            pltpu.async_copy(ones_v, cnt_sh.at[idx_v.at[j]],
                             sem_c[t], add=True)

        def pair_body(j2, _):
            for t in range(2):
                j = 2 * j2 + t

                @pl.when(j2 >= 1)
                def _(t=t):
                    drain_slot(t)

                fire_chunk(j, t)
            return 0

        lax.fori_loop(0, NCH // 2, pair_body, 0)
        drain_slot(0)
        drain_slot(1)
        fire_chunk(NCH - 1, 0)
        drain_slot(0)

        plsc.subcore_barrier()

        pltpu.sync_copy(acc_sh.at[pl.ds(row0, ROWS_PER_TILE)],
                        sums_out.at[cid, pl.ds(row0, ROWS_PER_TILE)])
        pltpu.sync_copy(cnt_sh.at[pl.ds(sid * CNT_PER_TILE, CNT_PER_TILE)],
                        cnt_out.at[cid, pl.ds(sid * CNT_PER_TILE, CNT_PER_TILE)])

    return sc_kernel(nbr_feat, idx2d)


def _tc_combine(self_feat, sums_p, cnt_p, W1, W2):
    BLK = 1024
    grid = (N_NODES + BLK - 1) // BLK

    def tc_kernel(self_ref, sums_ref, cnt_ref, w1_ref, w2_ref, out_ref):
        pid = pl.program_id(0)
        s = sums_ref[0] + sums_ref[1]
        c = (cnt_ref[0, pl.ds(pid * BLK, BLK)]
             + cnt_ref[1, pl.ds(pid * BLK, BLK)])
        denom = jnp.maximum(c, 1.0)
        agg = s * (1.0 / denom)[:, None]
        dn = (((1,), (1,)), ((), ()))
        out_ref[...] = (
            lax.dot_general(self_ref[...], w1_ref[...], dn,
                            preferred_element_type=jnp.float32)
            + lax.dot_general(agg, w2_ref[...], dn,
                              preferred_element_type=jnp.float32)
        )

    return pl.pallas_call(
        tc_kernel,
        out_shape=jax.ShapeDtypeStruct((N_NODES, D), jnp.float32),
        grid=(grid,),
        in_specs=[
            pl.BlockSpec((BLK, D), lambda i: (i, 0)),
            pl.BlockSpec((NC, BLK, D), lambda i: (0, i, 0)),
            pl.BlockSpec((NC, CNT_PAD), lambda i: (0, 0)),
            pl.BlockSpec((D, D), lambda i: (0, 0)),
            pl.BlockSpec((D, D), lambda i: (0, 0)),
        ],
        out_specs=pl.BlockSpec((BLK, D), lambda i: (i, 0)),
    )(self_feat, sums_p, cnt_p, W1, W2)


@jax.jit
def kernel(self_feat, nbr_feat, relation_src_indices, W):
    idx3d = relation_src_indices.astype(jnp.int32).reshape(
        NC * NS, IDX_ROWS, SCAT)
    sums_p, cnt_p = _sc_segment_sum(nbr_feat, idx3d)
    W1 = W[:, :D]
    W2 = W[:, D:]
    return _tc_combine(self_feat, sums_p, cnt_p, W1, W2)
